# Initial kernel scaffold; baseline (speedup 1.0000x reference)
import jax
import jax.numpy as jnp
from jax.experimental import pallas as pl


def kernel(h, e, q, Wg1, bg1, Wg2, bg2, Ws1, bs1, Ws2, bs2, edge_index, edge_batch):
    raise NotImplementedError("write your pallas kernel here")


# speedup vs baseline: 1.0753x; 1.0753x over previous
import functools
import jax
import jax.numpy as jnp
from jax import lax
from jax.experimental import pallas as pl
from jax.experimental.pallas import tpu as pltpu
from jax.experimental.pallas import tpu_sc as plsc

N_NODES = 10000
N_EDGES = 160000
NC, NS, L = 2, 16, 16
NW = NC * NS
NVREG = N_EDGES // L
VB = NVREG // NW
VREM = NVREG - VB * NW
DH = 256


def _mm_body(x_ref, w_ref, b_ref, o_ref):
    o_ref[...] = (
        jnp.dot(x_ref[...], w_ref[...], preferred_element_type=jnp.float32)
        + b_ref[...]
    )


def _mm(x, w, b, bm):
    m, k = x.shape
    n = w.shape[1]
    grid = m // bm
    return pl.pallas_call(
        _mm_body,
        grid=(grid,),
        in_specs=[
            pl.BlockSpec((bm, k), lambda i: (i, 0)),
            pl.BlockSpec((k, n), lambda i: (0, 0)),
            pl.BlockSpec((1, n), lambda i: (0, 0)),
        ],
        out_specs=pl.BlockSpec((bm, n), lambda i: (i, 0)),
        out_shape=jax.ShapeDtypeStruct((m, n), jnp.float32),
    )(x, w, b.reshape(1, n))


def _worker_range(wid):
    nv = VB + jnp.where(wid < VREM, 1, 0)
    start = wid * VB + jnp.minimum(wid, VREM)
    return start, nv


def _sc_edges_body(tsrc, tdst, qproj, eproj, srci, dsti, bati, w2cat, params,
                   ex_out, part_out,
                   si_v, di_v, bi_v, srows, drows, qrows, erows,
                   w2_v, par_v, sums_v, ex_v, rg_v, rs_v, sem):
    wid = lax.axis_index("s") * NC + lax.axis_index("c")
    start, nv = _worker_range(wid)
    pltpu.sync_copy(w2cat, w2_v)
    pltpu.sync_copy(params, par_v)
    pv = par_v[...]
    bg2 = pv[0]
    bs2 = pv[1]

    def zero_body(i, _):
        sums_v[pl.ds(i * L, L)] = jnp.zeros((L,), jnp.float32)
        return 0

    lax.fori_loop(0, N_NODES // L, zero_body, 0)

    lanes = lax.iota(jnp.int32, L)

    def blk(i, _):
        base = (start + i) * L
        pltpu.sync_copy(srci.at[pl.ds(base, L)], si_v)
        pltpu.sync_copy(dsti.at[pl.ds(base, L)], di_v)
        pltpu.sync_copy(bati.at[pl.ds(base, L)], bi_v)
        pltpu.async_copy(tsrc.at[si_v], srows, sem).wait()
        pltpu.async_copy(tdst.at[di_v], drows, sem).wait()
        pltpu.async_copy(qproj.at[bi_v], qrows, sem).wait()
        pltpu.sync_copy(eproj.at[pl.ds(base, L)], erows)
        for l in range(L):
            def dot_g(j, acc):
                a = srows[l, pl.ds(j * L, L)]
                b = drows[l, pl.ds(j * L, L)]
                c = qrows[l, pl.ds(j * L, L)]
                w = w2_v[pl.ds(j * L, L)]
                return acc + jnp.maximum(a + b + c, 0.0) * w

            def dot_s(j, acc):
                a = srows[l, pl.ds(DH + j * L, L)]
                b = drows[l, pl.ds(DH + j * L, L)]
                c = erows[l, pl.ds(j * L, L)]
                w = w2_v[pl.ds(DH + j * L, L)]
                return acc + jnp.maximum(a + b + c, 0.0) * w

            accg = lax.fori_loop(0, DH // L, dot_g, jnp.zeros((L,), jnp.float32))
            accs = lax.fori_loop(0, DH // L, dot_s, jnp.zeros((L,), jnp.float32))
            rg_v[pl.ds(l * L, L)] = accg
            rs_v[pl.ds(l * L, L)] = accs
        G = jnp.zeros((L,), jnp.float32) + bg2
        S = jnp.zeros((L,), jnp.float32) + bs2
        for c in range(L):
            G = G + plsc.load_gather(rg_v, [lanes * L + c])
            S = S + plsc.load_gather(rs_v, [lanes * L + c])
        gate = 1.0 / (1.0 + jnp.exp(-G))
        ex = jnp.exp(gate * S)
        ex_v[...] = ex
        pltpu.sync_copy(ex_v, ex_out.at[pl.ds(base, L)])
        dstv = di_v[...]
        plsc.addupdate_scatter(sums_v, [dstv], ex)
        return 0

    lax.fori_loop(0, nv, blk, 0)
    pltpu.sync_copy(sums_v, part_out.at[wid])


def _sc_norm_body(parts, dsti, exh, out, acc_v, tmp_v, di_v, ex_v, o_v, sem):
    wid = lax.axis_index("s") * NC + lax.axis_index("c")
    start, nv = _worker_range(wid)

    def zero_body(i, _):
        acc_v[pl.ds(i * L, L)] = jnp.zeros((L,), jnp.float32)
        return 0

    lax.fori_loop(0, N_NODES // L, zero_body, 0)
    for r in range(NW):
        pltpu.sync_copy(parts.at[r], tmp_v)

        def add_body(i, _):
            acc_v[pl.ds(i * L, L)] = acc_v[pl.ds(i * L, L)] + tmp_v[pl.ds(i * L, L)]
            return 0

        lax.fori_loop(0, N_NODES // L, add_body, 0)

    def blk(i, _):
        base = (start + i) * L
        pltpu.sync_copy(dsti.at[pl.ds(base, L)], di_v)
        pltpu.sync_copy(exh.at[pl.ds(base, L)], ex_v)
        s = plsc.load_gather(acc_v, [di_v[...]])
        o_v[...] = ex_v[...] / jnp.maximum(s, 1e-9)
        pltpu.sync_copy(o_v, out.at[pl.ds(base, L)])
        return 0

    lax.fori_loop(0, nv, blk, 0)


def kernel(h, e, q, Wg1, bg1, Wg2, bg2, Ws1, bs1, Ws2, bs2, edge_index, edge_batch):
    dh = h.shape[1]
    w_src = jnp.concatenate([Wg1[:dh], Ws1[:dh]], axis=1)
    w_dst = jnp.concatenate([Wg1[dh:2 * dh], Ws1[dh:2 * dh]], axis=1)
    zeros512 = jnp.zeros((2 * dh,), jnp.float32)
    tsrc = _mm(h, w_src, zeros512, 2000)
    tdst = _mm(h, w_dst, zeros512, 2000)
    qproj = _mm(q, Wg1[2 * dh:], bg1, q.shape[0])
    eproj = _mm(e, Ws1[2 * dh:], bs1, 2000)

    w2cat = jnp.concatenate([Wg2[:, 0], Ws2[:, 0]])
    params = jnp.zeros((16,), jnp.float32).at[0].set(bg2[0]).at[1].set(bs2[0])
    srci = edge_index[0]
    dsti = edge_index[1]

    mesh = plsc.VectorSubcoreMesh(core_axis_name="c", subcore_axis_name="s")
    cparams = pltpu.CompilerParams(needs_layout_passes=False)
    ex, parts = pl.kernel(
        _sc_edges_body,
        mesh=mesh,
        compiler_params=cparams,
        out_type=[
            jax.ShapeDtypeStruct((N_EDGES,), jnp.float32),
            jax.ShapeDtypeStruct((NW, N_NODES), jnp.float32),
        ],
        scratch_types=[
            pltpu.VMEM((L,), jnp.int32),
            pltpu.VMEM((L,), jnp.int32),
            pltpu.VMEM((L,), jnp.int32),
            pltpu.VMEM((L, 2 * dh), jnp.float32),
            pltpu.VMEM((L, 2 * dh), jnp.float32),
            pltpu.VMEM((L, dh), jnp.float32),
            pltpu.VMEM((L, dh), jnp.float32),
            pltpu.VMEM((2 * dh,), jnp.float32),
            pltpu.VMEM((16,), jnp.float32),
            pltpu.VMEM((N_NODES,), jnp.float32),
            pltpu.VMEM((L,), jnp.float32),
            pltpu.VMEM((L * L,), jnp.float32),
            pltpu.VMEM((L * L,), jnp.float32),
            pltpu.SemaphoreType.DMA,
        ],
    )(tsrc, tdst, qproj, eproj, srci, dsti, edge_batch, w2cat, params)

    out = pl.kernel(
        _sc_norm_body,
        mesh=mesh,
        compiler_params=cparams,
        out_type=jax.ShapeDtypeStruct((N_EDGES,), jnp.float32),
        scratch_types=[
            pltpu.VMEM((N_NODES,), jnp.float32),
            pltpu.VMEM((N_NODES,), jnp.float32),
            pltpu.VMEM((L,), jnp.int32),
            pltpu.VMEM((L,), jnp.float32),
            pltpu.VMEM((L,), jnp.float32),
            pltpu.SemaphoreType.DMA,
        ],
    )(parts, dsti, ex)
    return out


# speedup vs baseline: 1.7274x; 1.6064x over previous
import functools
import jax
import jax.numpy as jnp
from jax import lax
from jax.experimental import pallas as pl
from jax.experimental.pallas import tpu as pltpu
from jax.experimental.pallas import tpu_sc as plsc

N_NODES = 10000
N_EDGES = 160000
NC, NS, L = 2, 16, 16
NW = NC * NS
EB = 64
NG = EB // L
NBLK = N_EDGES // EB
VB = NBLK // NW
VREM = NBLK - VB * NW
DH = 256


def _mm_body(x_ref, w_ref, b_ref, o_ref):
    o_ref[...] = (
        jnp.dot(x_ref[...], w_ref[...], preferred_element_type=jnp.float32)
        + b_ref[...]
    )


def _mm(x, w, b, bm):
    m, k = x.shape
    n = w.shape[1]
    grid = m // bm
    return pl.pallas_call(
        _mm_body,
        grid=(grid,),
        in_specs=[
            pl.BlockSpec((bm, k), lambda i: (i, 0)),
            pl.BlockSpec((k, n), lambda i: (0, 0)),
            pl.BlockSpec((1, n), lambda i: (0, 0)),
        ],
        out_specs=pl.BlockSpec((bm, n), lambda i: (i, 0)),
        out_shape=jax.ShapeDtypeStruct((m, n), jnp.float32),
    )(x, w, b.reshape(1, n))


def _worker_range(wid):
    nv = VB + jnp.where(wid < VREM, 1, 0)
    start = wid * VB + jnp.minimum(wid, VREM)
    return start, nv


def _sc_edges_body(tsrc, tdst, qproj, eproj, srci, dsti, bati, w2cat, params,
                   ex_out, part_out,
                   si_v, di_v, bi_v, srows, drows, qrows, erows,
                   w2_v, par_v, sums_v, ex_v, rg_v, rs_v, sem):
    wid = lax.axis_index("s") * NC + lax.axis_index("c")
    start, nv = _worker_range(wid)
    pltpu.sync_copy(w2cat, w2_v)
    pltpu.sync_copy(params, par_v)
    pv = par_v[...]
    bg2 = pv[0]
    bs2 = pv[1]

    def zero_body(i, _):
        sums_v[pl.ds(i * L, L)] = jnp.zeros((L,), jnp.float32)
        return 0

    lax.fori_loop(0, N_NODES // L, zero_body, 0)

    lanes = lax.iota(jnp.int32, L)

    def blk(i, _):
        base = (start + i) * EB
        pltpu.sync_copy(srci.at[pl.ds(base, EB)], si_v)
        pltpu.sync_copy(dsti.at[pl.ds(base, EB)], di_v)
        pltpu.sync_copy(bati.at[pl.ds(base, EB)], bi_v)
        c1 = pltpu.async_copy(tsrc.at[si_v], srows, sem)
        c2 = pltpu.async_copy(tdst.at[di_v], drows, sem)
        c3 = pltpu.async_copy(qproj.at[bi_v], qrows, sem)
        c4 = pltpu.async_copy(eproj.at[pl.ds(base, EB)], erows, sem)
        c1.wait()
        c2.wait()
        c3.wait()
        c4.wait()
        for l in range(EB):
            def dot_g(j, acc):
                a = srows[l, pl.ds(j * L, L)]
                b = drows[l, pl.ds(j * L, L)]
                c = qrows[l, pl.ds(j * L, L)]
                w = w2_v[pl.ds(j * L, L)]
                return acc + jnp.maximum(a + b + c, 0.0) * w

            def dot_s(j, acc):
                a = srows[l, pl.ds(DH + j * L, L)]
                b = drows[l, pl.ds(DH + j * L, L)]
                c = erows[l, pl.ds(j * L, L)]
                w = w2_v[pl.ds(DH + j * L, L)]
                return acc + jnp.maximum(a + b + c, 0.0) * w

            accg = lax.fori_loop(0, DH // L, dot_g, jnp.zeros((L,), jnp.float32))
            accs = lax.fori_loop(0, DH // L, dot_s, jnp.zeros((L,), jnp.float32))
            rg_v[pl.ds(l * L, L)] = accg
            rs_v[pl.ds(l * L, L)] = accs
        for g in range(NG):
            G = jnp.zeros((L,), jnp.float32) + bg2
            S = jnp.zeros((L,), jnp.float32) + bs2
            gbase = g * L * L
            for c in range(L):
                G = G + plsc.load_gather(rg_v, [gbase + lanes * L + c])
                S = S + plsc.load_gather(rs_v, [gbase + lanes * L + c])
            gate = 1.0 / (1.0 + jnp.exp(-G))
            ex = jnp.exp(gate * S)
            ex_v[pl.ds(g * L, L)] = ex
            dstv = di_v[pl.ds(g * L, L)]
            plsc.addupdate_scatter(sums_v, [dstv], ex)
        pltpu.sync_copy(ex_v, ex_out.at[pl.ds(base, EB)])
        return 0

    lax.fori_loop(0, nv, blk, 0)
    pltpu.sync_copy(sums_v, part_out.at[wid])


def _sc_norm_body(parts, dsti, exh, out, acc_v, tmp_v, di_v, ex_v, o_v, sem):
    wid = lax.axis_index("s") * NC + lax.axis_index("c")
    start, nv = _worker_range(wid)

    def zero_body(i, _):
        acc_v[pl.ds(i * L, L)] = jnp.zeros((L,), jnp.float32)
        return 0

    lax.fori_loop(0, N_NODES // L, zero_body, 0)
    for r in range(NW):
        pltpu.sync_copy(parts.at[r], tmp_v)

        def add_body(i, _):
            acc_v[pl.ds(i * L, L)] = acc_v[pl.ds(i * L, L)] + tmp_v[pl.ds(i * L, L)]
            return 0

        lax.fori_loop(0, N_NODES // L, add_body, 0)

    def blk(i, _):
        base = (start + i) * EB
        pltpu.sync_copy(dsti.at[pl.ds(base, EB)], di_v)
        pltpu.sync_copy(exh.at[pl.ds(base, EB)], ex_v)
        for g in range(NG):
            s = plsc.load_gather(acc_v, [di_v[pl.ds(g * L, L)]])
            o_v[pl.ds(g * L, L)] = ex_v[pl.ds(g * L, L)] / jnp.maximum(s, 1e-9)
        pltpu.sync_copy(o_v, out.at[pl.ds(base, EB)])
        return 0

    lax.fori_loop(0, nv, blk, 0)


def kernel(h, e, q, Wg1, bg1, Wg2, bg2, Ws1, bs1, Ws2, bs2, edge_index, edge_batch):
    dh = h.shape[1]
    w_src = jnp.concatenate([Wg1[:dh], Ws1[:dh]], axis=1)
    w_dst = jnp.concatenate([Wg1[dh:2 * dh], Ws1[dh:2 * dh]], axis=1)
    zeros512 = jnp.zeros((2 * dh,), jnp.float32)
    tsrc = _mm(h, w_src, zeros512, 2000)
    tdst = _mm(h, w_dst, zeros512, 2000)
    qproj = _mm(q, Wg1[2 * dh:], bg1, q.shape[0])
    eproj = _mm(e, Ws1[2 * dh:], bs1, 2000)

    w2cat = jnp.concatenate([Wg2[:, 0], Ws2[:, 0]])
    params = jnp.zeros((16,), jnp.float32).at[0].set(bg2[0]).at[1].set(bs2[0])
    srci = edge_index[0]
    dsti = edge_index[1]

    mesh = plsc.VectorSubcoreMesh(core_axis_name="c", subcore_axis_name="s")
    cparams = pltpu.CompilerParams(needs_layout_passes=False)
    ex, parts = pl.kernel(
        _sc_edges_body,
        mesh=mesh,
        compiler_params=cparams,
        out_type=[
            jax.ShapeDtypeStruct((N_EDGES,), jnp.float32),
            jax.ShapeDtypeStruct((NW, N_NODES), jnp.float32),
        ],
        scratch_types=[
            pltpu.VMEM((EB,), jnp.int32),
            pltpu.VMEM((EB,), jnp.int32),
            pltpu.VMEM((EB,), jnp.int32),
            pltpu.VMEM((EB, 2 * dh), jnp.float32),
            pltpu.VMEM((EB, 2 * dh), jnp.float32),
            pltpu.VMEM((EB, dh), jnp.float32),
            pltpu.VMEM((EB, dh), jnp.float32),
            pltpu.VMEM((2 * dh,), jnp.float32),
            pltpu.VMEM((16,), jnp.float32),
            pltpu.VMEM((N_NODES,), jnp.float32),
            pltpu.VMEM((EB,), jnp.float32),
            pltpu.VMEM((EB * L,), jnp.float32),
            pltpu.VMEM((EB * L,), jnp.float32),
            pltpu.SemaphoreType.DMA,
        ],
    )(tsrc, tdst, qproj, eproj, srci, dsti, edge_batch, w2cat, params)

    out = pl.kernel(
        _sc_norm_body,
        mesh=mesh,
        compiler_params=cparams,
        out_type=jax.ShapeDtypeStruct((N_EDGES,), jnp.float32),
        scratch_types=[
            pltpu.VMEM((N_NODES,), jnp.float32),
            pltpu.VMEM((N_NODES,), jnp.float32),
            pltpu.VMEM((EB,), jnp.int32),
            pltpu.VMEM((EB,), jnp.float32),
            pltpu.VMEM((EB,), jnp.float32),
            pltpu.SemaphoreType.DMA,
        ],
    )(parts, dsti, ex)
    return out
